# baseline (speedup 1.0000x reference)
import functools

import jax
import jax.numpy as jnp
from jax.experimental import pallas as pl

B = 128
IN_DIM = 64
OUT_DIM = 32768
RB = 32
NRB = B // RB


def _kernel_body(x_ref, k_ref, w_ref, b_ref, ones_ref, out_ref):
    logits = jax.lax.dot_general(
        x_ref[...], w_ref[...], (((1,), (1,)), ((), ())),
        preferred_element_type=jnp.float32) + b_ref[...]
    k = k_ref[...]
    s = logits + k

    nsq = jax.lax.dot_general(
        s * s, ones_ref[...], (((1,), (0,)), ((), ())),
        preferred_element_type=jnp.float32)
    rnorm = 1.0 / jnp.maximum(jnp.sqrt(nsq), 1e-12)

    neg_inf = jnp.float32(-jnp.inf)
    m1 = jnp.max(logits, axis=1, keepdims=True)
    v2 = jnp.max(jnp.where(logits < m1, logits, neg_inf),
                 axis=1, keepdims=True)

    q = s * rnorm
    out_ref[...] = jnp.where(logits >= v2, q, k * q)


@functools.partial(jax.jit, static_argnames=())
def kernel(x, input_knowledge, W, b):
    b2 = b.reshape(1, OUT_DIM)
    ones = jnp.ones((OUT_DIM, 1), jnp.float32)
    return pl.pallas_call(
        _kernel_body,
        grid=(NRB,),
        in_specs=[
            pl.BlockSpec((RB, IN_DIM), lambda r: (r, 0)),
            pl.BlockSpec((RB, OUT_DIM), lambda r: (r, 0)),
            pl.BlockSpec((OUT_DIM, IN_DIM), lambda r: (0, 0)),
            pl.BlockSpec((1, OUT_DIM), lambda r: (0, 0)),
            pl.BlockSpec((OUT_DIM, 1), lambda r: (0, 0)),
        ],
        out_specs=pl.BlockSpec((RB, OUT_DIM), lambda r: (r, 0)),
        out_shape=jax.ShapeDtypeStruct((B, OUT_DIM), jnp.float32),
    )(x, input_knowledge, W, b2, ones)


# speedup vs baseline: 1.3803x; 1.3803x over previous
import functools

import jax
import jax.numpy as jnp
from jax.experimental import pallas as pl

B = 128
IN_DIM = 64
OUT_DIM = 32768
RB = 32
NRB = B // RB


def _kernel_body(x_ref, k_ref, w_ref, b_ref, out_ref):
    logits = jax.lax.dot_general(
        x_ref[...], w_ref[...], (((1,), (1,)), ((), ())),
        preferred_element_type=jnp.float32) + b_ref[...]
    k = k_ref[...]
    s = logits + k

    nsq = jnp.sum(s * s, axis=1, keepdims=True)
    rnorm = 1.0 / jnp.maximum(jnp.sqrt(nsq), 1e-12)

    neg_inf = jnp.float32(-jnp.inf)
    m1 = jnp.max(logits, axis=1, keepdims=True)
    v2 = jnp.max(jnp.where(logits < m1, logits, neg_inf),
                 axis=1, keepdims=True)

    q = s * rnorm
    out_ref[...] = jnp.where(logits >= v2, q, k * q)


@functools.partial(jax.jit, static_argnames=())
def kernel(x, input_knowledge, W, b):
    b2 = b.reshape(1, OUT_DIM)
    return pl.pallas_call(
        _kernel_body,
        grid=(NRB,),
        in_specs=[
            pl.BlockSpec((RB, IN_DIM), lambda r: (r, 0)),
            pl.BlockSpec((RB, OUT_DIM), lambda r: (r, 0)),
            pl.BlockSpec((OUT_DIM, IN_DIM), lambda r: (0, 0)),
            pl.BlockSpec((1, OUT_DIM), lambda r: (0, 0)),
        ],
        out_specs=pl.BlockSpec((RB, OUT_DIM), lambda r: (r, 0)),
        out_shape=jax.ShapeDtypeStruct((B, OUT_DIM), jnp.float32),
    )(x, input_knowledge, W, b2)
